# baseline (speedup 1.0000x reference)
import functools

import jax
import jax.numpy as jnp
from jax import lax
from jax.experimental import pallas as pl
from jax.experimental.pallas import tpu as pltpu
from jax.experimental.pallas import tpu_sc as plsc

HIDDEN = 4096
NUM_EXPERTS = 64
TOP_K = 8
TOKENS = 16384


_TC_BLOCK = 1024
_CHUNKS = 4
_CHUNK_TOKENS = TOKENS // _CHUNKS


def _scores_body(x_ref, wt_ref, out_ref):
    l = lax.dot_general(
        x_ref[...].astype(jnp.bfloat16), wt_ref[...].astype(jnp.bfloat16),
        (((1,), (0,)), ((), ())),
        preferred_element_type=jnp.float32,
    )
    m = jnp.max(l, axis=-1, keepdims=True)
    e = jnp.exp(l - m)
    out_ref[...] = e / jnp.sum(e, axis=-1, keepdims=True)


def _scores_tc(x, wt):
    n = x.shape[0]
    return pl.pallas_call(
        _scores_body,
        grid=(n // _TC_BLOCK,),
        in_specs=[
            pl.BlockSpec((_TC_BLOCK, HIDDEN), lambda i: (i, 0)),
            pl.BlockSpec((HIDDEN, NUM_EXPERTS), lambda i: (0, 0)),
        ],
        out_specs=pl.BlockSpec((_TC_BLOCK, NUM_EXPERTS), lambda i: (i, 0)),
        out_shape=jax.ShapeDtypeStruct((n, NUM_EXPERTS), jnp.float32),
    )(x, wt)


_NW = 32
_L = 16


def _topk_body(tpw, scores_hbm, w_hbm, e_hbm, sc_v, w_v, e_v):
    wid = lax.axis_index("s") * 2 + lax.axis_index("c")
    base = wid * tpw
    pltpu.sync_copy(scores_hbm.at[pl.ds(base * NUM_EXPERTS, tpw * NUM_EXPERTS)], sc_v)

    lanes = lax.iota(jnp.int32, _L)

    def group(g, carry):
        rows = g * _L + lanes
        sbase = rows * NUM_EXPERTS

        def expert(e, r):
            v = plsc.load_gather(sc_v, [sbase + e])
            vb = plsc.bitcast(v, jnp.uint32)
            vb = (vb & jnp.uint32(0xFFFFFFC0)) | (63 - e).astype(jnp.uint32)
            vp = plsc.bitcast(vb, jnp.float32)
            for j in range(TOP_K):
                hi = jnp.maximum(r[j], vp)
                vp = jnp.minimum(r[j], vp)
                r = r[:j] + (hi,) + r[j + 1:]
            return r

        r = lax.fori_loop(0, NUM_EXPERTS, expert,
                          (jnp.zeros((_L,), jnp.float32),) * TOP_K,
                          unroll=8)

        idxs = []
        vals = []
        for j in range(TOP_K):
            bits = plsc.bitcast(r[j], jnp.uint32)
            idx = (63 - (bits & jnp.uint32(63)).astype(jnp.int32))
            idxs.append(idx)
            vals.append(plsc.load_gather(sc_v, [sbase + idx]))
        total = vals[0]
        for j in range(1, TOP_K):
            total = total + vals[j]
        inv = 1.0 / total
        out_base = rows * TOP_K
        for j in range(TOP_K):
            plsc.store_scatter(w_v, [out_base + j], vals[j] * inv)
            plsc.store_scatter(e_v, [out_base + j], idxs[j])
        return carry

    lax.fori_loop(0, tpw // _L, group, 0)
    pltpu.sync_copy(w_v, w_hbm.at[pl.ds(base * TOP_K, tpw * TOP_K)])
    pltpu.sync_copy(e_v, e_hbm.at[pl.ds(base * TOP_K, tpw * TOP_K)])


def _topk_sc(scores):
    n = scores.shape[0]
    tpw = n // _NW
    w_flat, e_flat = pl.kernel(
        functools.partial(_topk_body, tpw),
        out_type=(
            jax.ShapeDtypeStruct((n * TOP_K,), jnp.float32),
            jax.ShapeDtypeStruct((n * TOP_K,), jnp.int32),
        ),
        mesh=plsc.VectorSubcoreMesh(core_axis_name="c", subcore_axis_name="s"),
        compiler_params=pltpu.CompilerParams(needs_layout_passes=False),
        scratch_types=[
            pltpu.VMEM((tpw * NUM_EXPERTS,), jnp.float32),
            pltpu.VMEM((tpw * TOP_K,), jnp.float32),
            pltpu.VMEM((tpw * TOP_K,), jnp.int32),
        ],
    )(scores.reshape(-1))
    return (w_flat.reshape(n, TOP_K), e_flat.reshape(n, TOP_K))


def kernel(x, W):
    wt = W.T
    scs, ws, es = [], [], []
    for c in range(_CHUNKS):
        xc = lax.slice(x, (c * _CHUNK_TOKENS, 0),
                       ((c + 1) * _CHUNK_TOKENS, HIDDEN))
        sc = _scores_tc(xc, wt)
        w, e = _topk_sc(sc)
        scs.append(sc)
        ws.append(w)
        es.append(e)
    return (jnp.concatenate(scs, 0), jnp.concatenate(ws, 0),
            jnp.concatenate(es, 0))


# speedup vs baseline: 2.8298x; 2.8298x over previous
import functools

import jax
import jax.numpy as jnp
from jax import lax
from jax.experimental import pallas as pl
from jax.experimental.pallas import tpu as pltpu
from jax.experimental.pallas import tpu_sc as plsc

HIDDEN = 4096
NUM_EXPERTS = 64
TOP_K = 8
TOKENS = 16384


_TC_BLOCK = 1024


def _scores_body(x_ref, wt_ref, out_ref, out_t_ref):
    l = lax.dot_general(
        x_ref[...].astype(jnp.bfloat16), wt_ref[...].astype(jnp.bfloat16),
        (((1,), (0,)), ((), ())),
        preferred_element_type=jnp.float32,
    )
    m = jnp.max(l, axis=-1, keepdims=True)
    e = jnp.exp(l - m)
    s = e / jnp.sum(e, axis=-1, keepdims=True)
    out_ref[...] = s
    out_t_ref[...] = s.T


def _scores_tc(x, wt):
    n = x.shape[0]
    return pl.pallas_call(
        _scores_body,
        grid=(n // _TC_BLOCK,),
        in_specs=[
            pl.BlockSpec((_TC_BLOCK, HIDDEN), lambda i: (i, 0)),
            pl.BlockSpec((HIDDEN, NUM_EXPERTS), lambda i: (0, 0)),
        ],
        out_specs=[
            pl.BlockSpec((_TC_BLOCK, NUM_EXPERTS), lambda i: (i, 0)),
            pl.BlockSpec((NUM_EXPERTS, _TC_BLOCK), lambda i: (0, i)),
        ],
        out_shape=[
            jax.ShapeDtypeStruct((n, NUM_EXPERTS), jnp.float32),
            jax.ShapeDtypeStruct((NUM_EXPERTS, n), jnp.float32),
        ],
    )(x, wt)


_NW = 32
_L = 16


def _topk_body(tpw, scores_t_hbm, w_hbm, e_hbm, sc_v, w_v, e_v):
    wid = lax.axis_index("s") * 2 + lax.axis_index("c")
    base = wid * tpw
    pltpu.sync_copy(scores_t_hbm.at[:, pl.ds(base, tpw)], sc_v)

    def group(g, carry):
        col = g * _L

        r = (jnp.zeros((_L,), jnp.float32),) * TOP_K
        for e in range(NUM_EXPERTS):
            v = sc_v[e, pl.ds(col, _L)]
            vb = plsc.bitcast(v, jnp.uint32)
            vp = plsc.bitcast(
                (vb & jnp.uint32(0xFFFFFFC0)) | jnp.uint32(63 - e),
                jnp.float32)
            for j in range(TOP_K):
                hi = jnp.maximum(r[j], vp)
                vp = jnp.minimum(r[j], vp)
                r = r[:j] + (hi,) + r[j + 1:]

        bits = [plsc.bitcast(r[j], jnp.uint32) for j in range(TOP_K)]
        vals = [plsc.bitcast(b & jnp.uint32(0xFFFFFFC0), jnp.float32)
                for b in bits]
        total = vals[0]
        for j in range(1, TOP_K):
            total = total + vals[j]
        inv = 1.0 / total
        for j in range(TOP_K):
            w_v[j, pl.ds(col, _L)] = vals[j] * inv
            e_v[j, pl.ds(col, _L)] = (
                63 - (bits[j] & jnp.uint32(63)).astype(jnp.int32))
        return carry

    lax.fori_loop(0, tpw // _L, group, 0)
    pltpu.sync_copy(w_v, w_hbm.at[:, pl.ds(base, tpw)])
    pltpu.sync_copy(e_v, e_hbm.at[:, pl.ds(base, tpw)])


def _topk_sc(scores_t):
    n = scores_t.shape[1]
    tpw = n // _NW
    w_t, e_t = pl.kernel(
        functools.partial(_topk_body, tpw),
        out_type=(
            jax.ShapeDtypeStruct((TOP_K, n), jnp.float32),
            jax.ShapeDtypeStruct((TOP_K, n), jnp.int32),
        ),
        mesh=plsc.VectorSubcoreMesh(core_axis_name="c", subcore_axis_name="s"),
        compiler_params=pltpu.CompilerParams(needs_layout_passes=False),
        scratch_types=[
            pltpu.VMEM((NUM_EXPERTS, tpw), jnp.float32),
            pltpu.VMEM((TOP_K, tpw), jnp.float32),
            pltpu.VMEM((TOP_K, tpw), jnp.int32),
        ],
    )(scores_t)
    return w_t, e_t


def kernel(x, W):
    scores, scores_t = _scores_tc(x, W.T)
    w_t, e_t = _topk_sc(scores_t)
    return (scores, w_t.T, e_t.T)
